# baseline (speedup 1.0000x reference)
import functools

import jax
import jax.numpy as jnp
from jax import lax
from jax.experimental import pallas as pl
from jax.experimental.pallas import tpu as pltpu
from jax.experimental.pallas import tpu_sc as plsc

_N_DRUG = 218
_N_CIR = 271
_N = _N_DRUG + _N_CIR
_HID = 128
_HEADS = 4
_NP = 512
_DUMMY = (_NP - 1) * _NP

_E_HET = 20000
_E_CIR = 8000
_HET_PER = 1248
_HET_LAST = _E_HET - 15 * _HET_PER
_CIR_PER = 504
_CIR_LAST = _E_CIR - 15 * _CIR_PER
_HET_SLOTS = 1280
_CIR_SLOTS = 512

_HET_ROWS = 496
_CIR_ROWS = 288
_HET_PW = _HET_ROWS * _NP // 16
_CIR_PW = _CIR_ROWS * _NP // 16


_CIR_BASE = _NP * _NP
_DUMMY_CIR = _CIR_BASE + 287 * _NP
_CWORDS = _CIR_BASE + _CIR_ROWS * _NP
_ZPW = _CWORDS // 16
_HPW = _HET_ROWS * _NP // 16
_CPW = _CIR_ROWS * _NP // 16
_H_PER = 2496
_C_PER = _E_CIR // 8
_H_SLOTS = 2560
_C_SLOTS = 1024
_H_LAST = _E_HET - 7 * _H_PER
assert _H_LAST % 8 == 0 and _H_LAST <= _H_SLOTS and _C_PER % 8 == 0


def _count_body(se_h, ce_h, out_het, out_cir,
                src_v, dst_v, flat_v, ones_v, buf_v, c_sh, sem, sem_e):
    s = lax.axis_index("s")

    zeros16 = jnp.zeros((16,), jnp.float32)
    ones16 = jnp.ones((16,), jnp.float32)
    iota16 = lax.iota(jnp.int32, 16)

    def _zb(i, carry):
        buf_v[pl.ds(i * 16, 16)] = zeros16
        return carry
    lax.fori_loop(0, _ZPW // 16, _zb, 0)
    zero_dma = pltpu.async_copy(buf_v.at[pl.ds(0, _ZPW)],
                                c_sh.at[pl.ds(s * _ZPW, _ZPW)], sem)
    for i in range(8):
        ones_v[pl.ds(i * 16, 16)] = ones16

    @pl.when(s < 7)
    def _():
        a = pltpu.async_copy(se_h.at[pl.ds(s * _H_PER, _H_PER)],
                             src_v.at[pl.ds(0, _H_PER)], sem_e)
        b = pltpu.async_copy(se_h.at[pl.ds(_E_HET + s * _H_PER, _H_PER)],
                             dst_v.at[pl.ds(0, _H_PER)], sem_e)
        a.wait()
        b.wait()

    @pl.when(s == 7)
    def _():
        a = pltpu.async_copy(se_h.at[pl.ds(7 * _H_PER, _H_LAST)],
                             src_v.at[pl.ds(0, _H_LAST)], sem_e)
        b = pltpu.async_copy(se_h.at[pl.ds(_E_HET + 7 * _H_PER, _H_LAST)],
                             dst_v.at[pl.ds(0, _H_LAST)], sem_e)
        a.wait()
        b.wait()

    @pl.when(s >= 8)
    def _():
        off = (s - 8) * _C_PER
        a = pltpu.async_copy(ce_h.at[pl.ds(off, _C_PER)],
                             src_v.at[pl.ds(0, _C_PER)], sem_e)
        b = pltpu.async_copy(ce_h.at[pl.ds(_E_CIR + off, _C_PER)],
                             dst_v.at[pl.ds(0, _C_PER)], sem_e)
        a.wait()
        b.wait()

    def _flat(nchunks, vc, base, dummy):
        for r in range(nchunks):
            def _fb(i, carry):
                off = r * 128 + i * 16
                sv = src_v[pl.ds(off, 16)]
                dv = dst_v[pl.ds(off, 16)]
                fl = jnp.where(iota16 + off < vc,
                               base + dv * _NP + sv, dummy)
                flat_v[r, pl.ds(i * 16, 16)] = fl
                return carry
            lax.fori_loop(0, 8, _fb, 0)

    @pl.when(s < 8)
    def _():
        _flat(_H_SLOTS // 128, jnp.where(s == 7, _H_LAST, _H_PER),
              0, _DUMMY)

    @pl.when(s >= 8)
    def _():
        _flat(_C_SLOTS // 128, _C_PER, _CIR_BASE, _DUMMY_CIR)

    zero_dma.wait()
    plsc.subcore_barrier()

    @pl.when(s < 8)
    def _():
        hs = [pltpu.async_copy(ones_v.at[pl.ds(0, 128)],
                               c_sh.at[flat_v.at[r]], sem, add=True)
              for r in range(_H_SLOTS // 128)]
        for h in hs:
            h.wait()

    @pl.when(s >= 8)
    def _():
        hs = [pltpu.async_copy(ones_v.at[pl.ds(0, 128)],
                               c_sh.at[flat_v.at[r]], sem, add=True)
              for r in range(_C_SLOTS // 128)]
        for h in hs:
            h.wait()

    plsc.subcore_barrier()
    a = pltpu.async_copy(c_sh.at[pl.ds(s * _HPW, _HPW)],
                         out_het.at[pl.ds(s * _HPW, _HPW)], sem_e)
    b = pltpu.async_copy(c_sh.at[pl.ds(_CIR_BASE + s * _CPW, _CPW)],
                         out_cir.at[pl.ds(s * _CPW, _CPW)], sem_e)
    a.wait()
    b.wait()


_count_kernel = functools.partial(
    pl.kernel,
    mesh=plsc.VectorSubcoreMesh(core_axis_name="c", subcore_axis_name="s",
                                num_cores=1),
    out_type=[jax.ShapeDtypeStruct((_HET_ROWS * _NP,), jnp.float32),
              jax.ShapeDtypeStruct((_CIR_ROWS * _NP,), jnp.float32)],
    scratch_types=[
        pltpu.VMEM((_H_SLOTS,), jnp.int32),
        pltpu.VMEM((_H_SLOTS,), jnp.int32),
        pltpu.VMEM((_H_SLOTS // 128, 128), jnp.int32),
        pltpu.VMEM((128,), jnp.float32),
        pltpu.VMEM((_ZPW,), jnp.float32),
        pltpu.VMEM_SHARED((_CWORDS,), jnp.float32),
        pltpu.SemaphoreType.DMA,
        pltpu.SemaphoreType.DMA,
    ],
)(_count_body)


def _dot(a, b, dims=((1,), (1,))):
    return lax.dot_general(
        a, b, (dims, ((), ())),
        precision=lax.Precision.DEFAULT,
        preferred_element_type=jnp.float32)


def _elu(out):
    return jnp.where(out > 0, out, jnp.exp(jnp.minimum(out, 0.0)) - 1.0)


def _attend(h, es, ed, valid, cv):
    e = ed + es
    e = jnp.where(e > 0, e, 0.2 * e)
    m = jnp.max(jnp.where(valid, e, -1e30), axis=1, keepdims=True)
    mm = jnp.where(m > -1e29, m, 0.0)
    p = jnp.where(valid, cv * jnp.exp(e - mm), 0.0)
    den = jnp.sum(p, axis=1, keepdims=True)
    alpha = p / (den + 1e-16)
    return _dot(alpha, h, dims=((1,), (0,)))


def _gat_dense(xv, valid, cv, w, asrc, adst, b, wres):
    acc = 0.0
    for hd in range(_HEADS):
        h = _dot(xv, w[hd], dims=((1,), (0,)))
        es = _dot(asrc[hd].reshape(1, _HID), h)
        ed = _dot(h, adst[hd].reshape(1, _HID))
        acc = acc + _attend(h, es, ed, valid, cv)
    out = acc * 0.25 + b.reshape(1, _HID) + _dot(xv, wres, dims=((1,), (0,)))
    return _elu(out)


def _pre_branch(xv, projw, projb, w0, as0, ad0, wr0):
    o0 = _dot(xv, projw, dims=((1,), (0,))) + projb.reshape(1, _HID)
    hs, ess, eds = [], [], []
    for hd in range(_HEADS):
        h = _dot(xv, w0[hd], dims=((1,), (0,)))
        hs.append(h.reshape(1, *h.shape))
        ess.append(_dot(as0[hd].reshape(1, _HID), h))
        eds.append(_dot(h, ad0[hd].reshape(1, _HID)))
    h_all = jnp.concatenate(hs, axis=0)
    es_all = jnp.concatenate(ess, axis=0)
    ed_all = jnp.concatenate(eds, axis=1)
    xr = _dot(xv, wr0, dims=((1,), (0,)))
    return o0, h_all, es_all, ed_all, xr


def _pre_body(x_ref, xc_ref, projw_ref, projb_ref, w0_ref, as0_ref, ad0_ref,
              wr0_ref, projwc_ref, projbc_ref, wc0_ref, asc0_ref, adc0_ref,
              wrc0_ref,
              o0_ref, h0_ref, es0_ref, ed0_ref, xr0_ref,
              o0c_ref, h0c_ref, es0c_ref, ed0c_ref, xrc_ref):
    o0, h0, es0, ed0, xr0 = _pre_branch(
        x_ref[...], projw_ref[...], projb_ref[...], w0_ref[...],
        as0_ref[...], ad0_ref[...], wr0_ref[...])
    o0_ref[...], h0_ref[...], es0_ref[...] = o0, h0, es0
    ed0_ref[...], xr0_ref[...] = ed0, xr0
    o0c, h0c, es0c, ed0c, xrc = _pre_branch(
        xc_ref[...], projwc_ref[...], projbc_ref[...], wc0_ref[...],
        asc0_ref[...], adc0_ref[...], wrc0_ref[...])
    o0c_ref[...], h0c_ref[...], es0c_ref[...] = o0c, h0c, es0c
    ed0c_ref[...], xrc_ref[...] = ed0c, xrc


def _branch(n, cfull, o0, h0, es0, ed0, xr0, b0,
            w1, as1, ad1, b1, wr1, cnn3, cnnb):
    cv = lax.slice(cfull, (0, 0), (n, n))
    valid = cv > 0.0
    acc = 0.0
    for hd in range(_HEADS):
        es = lax.slice(es0, (hd, 0), (hd + 1, n))
        ed = lax.slice(ed0, (0, hd), (n, hd + 1))
        acc = acc + _attend(h0[hd], es, ed, valid, cv)
    h1 = _elu(acc * 0.25 + b0.reshape(1, _HID) + xr0)
    h2 = _gat_dense(h1, valid, cv, w1, as1, ad1, b1, wr1)
    emb = (_dot(o0, cnn3[:, 0, :]) + _dot(h1, cnn3[:, 1, :])
           + _dot(h2, cnn3[:, 2, :]) + cnnb.reshape(1, _HID))
    return emb


def _dense_body(c_ref, cc_ref,
                o0_ref, h0_ref, es0_ref, ed0_ref, xr0_ref, b0_ref,
                w1_ref, as1_ref, ad1_ref, b1_ref, wr1_ref,
                cnn_ref, cnnb_ref,
                o0c_ref, h0c_ref, es0c_ref, ed0c_ref, xrc_ref, bc0_ref,
                wc1_ref, asc1_ref, adc1_ref, bc1_ref, wrc1_ref,
                cnnc_ref, cnnbc_ref,
                decw_ref, ret_ref, ass_ref):
    emb_het = _branch(_N, c_ref[...], o0_ref[...], h0_ref[...], es0_ref[...],
                      ed0_ref[...], xr0_ref[...], b0_ref[...], w1_ref[...],
                      as1_ref[...], ad1_ref[...], b1_ref[...], wr1_ref[...],
                      cnn_ref[...], cnnb_ref[...])
    emb_cir = _branch(_N_CIR, cc_ref[...], o0c_ref[...], h0c_ref[...],
                      es0c_ref[...], ed0c_ref[...], xrc_ref[...], bc0_ref[...],
                      wc1_ref[...], asc1_ref[...], adc1_ref[...], bc1_ref[...],
                      wrc1_ref[...], cnnc_ref[...], cnnbc_ref[...])
    drug = lax.slice(emb_het, (0, 0), (_N_DRUG, _HID))
    cir_het = lax.slice(emb_het, (_N_DRUG, 0), (_N, _HID))
    ass = 0.5 * (cir_het + emb_cir)
    t = _dot(drug, decw_ref[...], dims=((1,), (0,)))
    logits = _dot(t, ass)
    ret_ref[...] = 1.0 / (1.0 + jnp.exp(-logits))
    ass_ref[...] = ass


def kernel(x, edge_idx, x_cir, edge_idx_cir, params):
    p = params
    se = edge_idx.astype(jnp.int32).reshape(2 * _E_HET)
    ce = edge_idx_cir.astype(jnp.int32).reshape(2 * _E_CIR)
    c_het_flat, c_cir_flat = _count_kernel(se, ce)
    c_het = c_het_flat.reshape(_HET_ROWS, _NP)
    c_cir = c_cir_flat.reshape(_CIR_ROWS, _NP)
    cnn3 = p['cnn_het_W'].reshape(_HID, 3, _HID)
    cnn3c = p['cnn_cir_W'].reshape(_HID, 3, _HID)

    f32 = jnp.float32
    pre = pl.pallas_call(
        _pre_body,
        out_shape=[jax.ShapeDtypeStruct((_N, _HID), f32),
                   jax.ShapeDtypeStruct((_HEADS, _N, _HID), f32),
                   jax.ShapeDtypeStruct((_HEADS, _N), f32),
                   jax.ShapeDtypeStruct((_N, _HEADS), f32),
                   jax.ShapeDtypeStruct((_N, _HID), f32),
                   jax.ShapeDtypeStruct((_N_CIR, _HID), f32),
                   jax.ShapeDtypeStruct((_HEADS, _N_CIR, _HID), f32),
                   jax.ShapeDtypeStruct((_HEADS, _N_CIR), f32),
                   jax.ShapeDtypeStruct((_N_CIR, _HEADS), f32),
                   jax.ShapeDtypeStruct((_N_CIR, _HID), f32)],
    )(x, x_cir,
      p['proj_W'], p['proj_b'],
      p['conv0']['W'], p['conv0']['a_src'], p['conv0']['a_dst'],
      p['conv0']['W_res'],
      p['proj_cir_W'], p['proj_cir_b'],
      p['convc0']['W'], p['convc0']['a_src'], p['convc0']['a_dst'],
      p['convc0']['W_res'])
    o0, h0, es0, ed0, xr0, o0c, h0c, es0c, ed0c, xrc = pre

    ret, ass = pl.pallas_call(
        _dense_body,
        out_shape=[jax.ShapeDtypeStruct((_N_DRUG, _N_CIR), f32),
                   jax.ShapeDtypeStruct((_N_CIR, _HID), f32)],
    )(c_het, c_cir,
      o0, h0, es0, ed0, xr0, p['conv0']['b'],
      p['conv1']['W'], p['conv1']['a_src'], p['conv1']['a_dst'],
      p['conv1']['b'], p['conv1']['W_res'],
      cnn3, p['cnn_het_b'],
      o0c, h0c, es0c, ed0c, xrc, p['convc0']['b'],
      p['convc1']['W'], p['convc1']['a_src'], p['convc1']['a_dst'],
      p['convc1']['b'], p['convc1']['W_res'],
      cnn3c, p['cnn_cir_b'],
      p['dec_W'])
    return (ret, ass)


# speedup vs baseline: 1.0672x; 1.0672x over previous
import functools

import jax
import jax.numpy as jnp
from jax import lax
from jax.experimental import pallas as pl
from jax.experimental.pallas import tpu as pltpu
from jax.experimental.pallas import tpu_sc as plsc

_N_DRUG = 218
_N_CIR = 271
_N = _N_DRUG + _N_CIR
_HID = 128
_HEADS = 4
_NP = 512
_DUMMY = (_NP - 1) * _NP

_E_HET = 20000
_E_CIR = 8000
_HET_PER = 1248
_HET_LAST = _E_HET - 15 * _HET_PER
_CIR_PER = 504
_CIR_LAST = _E_CIR - 15 * _CIR_PER
_HET_SLOTS = 1280
_CIR_SLOTS = 512

_HET_ROWS = 496
_CIR_ROWS = 288
_HET_PW = _HET_ROWS * _NP // 16
_CIR_PW = _CIR_ROWS * _NP // 16


def _count_body(se_h, ce_h, out_het, out_cir,
                src_v, dst_v, flat_v, ones_v, buf_v, c_sh, sem, sem_e):
    c = lax.axis_index("c")
    s = lax.axis_index("s")

    zeros16 = jnp.zeros((16,), jnp.float32)
    ones16 = jnp.ones((16,), jnp.float32)
    iota16 = lax.iota(jnp.int32, 16)

    def _zb(i, carry):
        buf_v[pl.ds(i * 16, 16)] = zeros16
        return carry
    lax.fori_loop(0, _HET_PW // 16, _zb, 0)
    for i in range(8):
        ones_v[pl.ds(i * 16, 16)] = ones16

    def _build(e_h, ne, per, last, slots, nchunks, pw):
        zero_dma = pltpu.async_copy(buf_v.at[pl.ds(0, pw)],
                                    c_sh.at[pl.ds(s * pw, pw)], sem)

        @pl.when(s < 15)
        def _():
            a = pltpu.async_copy(e_h.at[pl.ds(s * per, per)],
                                 src_v.at[pl.ds(0, per)], sem_e)
            b = pltpu.async_copy(e_h.at[pl.ds(ne + s * per, per)],
                                 dst_v.at[pl.ds(0, per)], sem_e)
            a.wait()
            b.wait()

        @pl.when(s == 15)
        def _():
            a = pltpu.async_copy(e_h.at[pl.ds(15 * per, last)],
                                 src_v.at[pl.ds(0, last)], sem_e)
            b = pltpu.async_copy(e_h.at[pl.ds(ne + 15 * per, last)],
                                 dst_v.at[pl.ds(0, last)], sem_e)
            a.wait()
            b.wait()

        vc = jnp.where(s == 15, last, per)
        for r in range(nchunks):
            def _fb(i, carry):
                off = r * 128 + i * 16
                sv = src_v[pl.ds(off, 16)]
                dv = dst_v[pl.ds(off, 16)]
                fl = jnp.where(iota16 + off < vc, dv * _NP + sv, _DUMMY)
                flat_v[r, pl.ds(i * 16, 16)] = fl
                return carry
            lax.fori_loop(0, 8, _fb, 0)
        zero_dma.wait()
        plsc.subcore_barrier()
        handles = [pltpu.async_copy(ones_v.at[pl.ds(0, 128)],
                                    c_sh.at[flat_v.at[r]], sem, add=True)
                   for r in range(nchunks)]
        for h in handles:
            h.wait()
        plsc.subcore_barrier()

    @pl.when(c == 0)
    def _():
        _build(se_h, _E_HET, _HET_PER, _HET_LAST, _HET_SLOTS,
               _HET_SLOTS // 128, _HET_PW)
        pltpu.sync_copy(c_sh.at[pl.ds(s * _HET_PW, _HET_PW)],
                        out_het.at[pl.ds(s * _HET_PW, _HET_PW)])

    @pl.when(c == 1)
    def _():
        _build(ce_h, _E_CIR, _CIR_PER, _CIR_LAST, _CIR_SLOTS,
               _CIR_SLOTS // 128, _CIR_PW)
        pltpu.sync_copy(c_sh.at[pl.ds(s * _CIR_PW, _CIR_PW)],
                        out_cir.at[pl.ds(s * _CIR_PW, _CIR_PW)])


_count_kernel = functools.partial(
    pl.kernel,
    mesh=plsc.VectorSubcoreMesh(core_axis_name="c", subcore_axis_name="s"),
    out_type=[jax.ShapeDtypeStruct((_HET_ROWS * _NP,), jnp.float32),
              jax.ShapeDtypeStruct((_CIR_ROWS * _NP,), jnp.float32)],
    scratch_types=[
        pltpu.VMEM((_HET_SLOTS,), jnp.int32),
        pltpu.VMEM((_HET_SLOTS,), jnp.int32),
        pltpu.VMEM((_HET_SLOTS // 128, 128), jnp.int32),
        pltpu.VMEM((128,), jnp.float32),
        pltpu.VMEM((_HET_PW,), jnp.float32),
        pltpu.VMEM_SHARED((_NP * _NP,), jnp.float32),
        pltpu.SemaphoreType.DMA,
        pltpu.SemaphoreType.DMA,
    ],
)(_count_body)


def _dot(a, b, dims=((1,), (1,))):
    return lax.dot_general(
        a, b, (dims, ((), ())),
        precision=lax.Precision.DEFAULT,
        preferred_element_type=jnp.float32)


def _elu(out):
    return jnp.where(out > 0, out, jnp.exp(jnp.minimum(out, 0.0)) - 1.0)


def _attend(h, es, ed, valid, cv):
    e = ed + es
    e = jnp.where(e > 0, e, 0.2 * e)
    m = jnp.max(jnp.where(valid, e, -1e30), axis=1, keepdims=True)
    mm = jnp.where(m > -1e29, m, 0.0)
    p = jnp.where(valid, cv * jnp.exp(e - mm), 0.0)
    den = jnp.sum(p, axis=1, keepdims=True)
    alpha = p / (den + 1e-16)
    return _dot(alpha, h, dims=((1,), (0,)))


def _gat_dense(xv, valid, cv, w, asrc, adst, b, wres):
    acc = 0.0
    for hd in range(_HEADS):
        h = _dot(xv, w[hd], dims=((1,), (0,)))
        es = _dot(asrc[hd].reshape(1, _HID), h)
        ed = _dot(h, adst[hd].reshape(1, _HID))
        acc = acc + _attend(h, es, ed, valid, cv)
    out = acc * 0.25 + b.reshape(1, _HID) + _dot(xv, wres, dims=((1,), (0,)))
    return _elu(out)


def _pre_branch(xv, projw, projb, w0, as0, ad0, wr0):
    o0 = _dot(xv, projw, dims=((1,), (0,))) + projb.reshape(1, _HID)
    hs, ess, eds = [], [], []
    for hd in range(_HEADS):
        h = _dot(xv, w0[hd], dims=((1,), (0,)))
        hs.append(h.reshape(1, *h.shape))
        ess.append(_dot(as0[hd].reshape(1, _HID), h))
        eds.append(_dot(h, ad0[hd].reshape(1, _HID)))
    h_all = jnp.concatenate(hs, axis=0)
    es_all = jnp.concatenate(ess, axis=0)
    ed_all = jnp.concatenate(eds, axis=1)
    xr = _dot(xv, wr0, dims=((1,), (0,)))
    return o0, h_all, es_all, ed_all, xr


def _pre_body(x_ref, xc_ref, projw_ref, projb_ref, w0_ref, as0_ref, ad0_ref,
              wr0_ref, projwc_ref, projbc_ref, wc0_ref, asc0_ref, adc0_ref,
              wrc0_ref,
              o0_ref, h0_ref, es0_ref, ed0_ref, xr0_ref,
              o0c_ref, h0c_ref, es0c_ref, ed0c_ref, xrc_ref):
    o0, h0, es0, ed0, xr0 = _pre_branch(
        x_ref[...], projw_ref[...], projb_ref[...], w0_ref[...],
        as0_ref[...], ad0_ref[...], wr0_ref[...])
    o0_ref[...], h0_ref[...], es0_ref[...] = o0, h0, es0
    ed0_ref[...], xr0_ref[...] = ed0, xr0
    o0c, h0c, es0c, ed0c, xrc = _pre_branch(
        xc_ref[...], projwc_ref[...], projbc_ref[...], wc0_ref[...],
        asc0_ref[...], adc0_ref[...], wrc0_ref[...])
    o0c_ref[...], h0c_ref[...], es0c_ref[...] = o0c, h0c, es0c
    ed0c_ref[...], xrc_ref[...] = ed0c, xrc


def _branch(n, cfull, o0, h0, es0, ed0, xr0, b0,
            w1, as1, ad1, b1, wr1, cnn3, cnnb):
    cv = lax.slice(cfull, (0, 0), (n, n))
    valid = cv > 0.0
    acc = 0.0
    for hd in range(_HEADS):
        es = lax.slice(es0, (hd, 0), (hd + 1, n))
        ed = lax.slice(ed0, (0, hd), (n, hd + 1))
        acc = acc + _attend(h0[hd], es, ed, valid, cv)
    h1 = _elu(acc * 0.25 + b0.reshape(1, _HID) + xr0)
    h2 = _gat_dense(h1, valid, cv, w1, as1, ad1, b1, wr1)
    emb = (_dot(o0, cnn3[:, 0, :]) + _dot(h1, cnn3[:, 1, :])
           + _dot(h2, cnn3[:, 2, :]) + cnnb.reshape(1, _HID))
    return emb


def _dense_body(c_ref, cc_ref,
                o0_ref, h0_ref, es0_ref, ed0_ref, xr0_ref, b0_ref,
                w1_ref, as1_ref, ad1_ref, b1_ref, wr1_ref,
                cnn_ref, cnnb_ref,
                o0c_ref, h0c_ref, es0c_ref, ed0c_ref, xrc_ref, bc0_ref,
                wc1_ref, asc1_ref, adc1_ref, bc1_ref, wrc1_ref,
                cnnc_ref, cnnbc_ref,
                decw_ref, ret_ref, ass_ref):
    emb_het = _branch(_N, c_ref[...], o0_ref[...], h0_ref[...], es0_ref[...],
                      ed0_ref[...], xr0_ref[...], b0_ref[...], w1_ref[...],
                      as1_ref[...], ad1_ref[...], b1_ref[...], wr1_ref[...],
                      cnn_ref[...], cnnb_ref[...])
    emb_cir = _branch(_N_CIR, cc_ref[...], o0c_ref[...], h0c_ref[...],
                      es0c_ref[...], ed0c_ref[...], xrc_ref[...], bc0_ref[...],
                      wc1_ref[...], asc1_ref[...], adc1_ref[...], bc1_ref[...],
                      wrc1_ref[...], cnnc_ref[...], cnnbc_ref[...])
    drug = lax.slice(emb_het, (0, 0), (_N_DRUG, _HID))
    cir_het = lax.slice(emb_het, (_N_DRUG, 0), (_N, _HID))
    ass = 0.5 * (cir_het + emb_cir)
    t = _dot(drug, decw_ref[...], dims=((1,), (0,)))
    logits = _dot(t, ass)
    ret_ref[...] = 1.0 / (1.0 + jnp.exp(-logits))
    ass_ref[...] = ass


def kernel(x, edge_idx, x_cir, edge_idx_cir, params):
    p = params
    se = edge_idx.astype(jnp.int32).reshape(2 * _E_HET)
    ce = edge_idx_cir.astype(jnp.int32).reshape(2 * _E_CIR)
    c_het_flat, c_cir_flat = _count_kernel(se, ce)
    c_het = c_het_flat.reshape(_HET_ROWS, _NP)
    c_cir = c_cir_flat.reshape(_CIR_ROWS, _NP)
    cnn3 = p['cnn_het_W'].reshape(_HID, 3, _HID)
    cnn3c = p['cnn_cir_W'].reshape(_HID, 3, _HID)

    f32 = jnp.float32
    pre = pl.pallas_call(
        _pre_body,
        out_shape=[jax.ShapeDtypeStruct((_N, _HID), f32),
                   jax.ShapeDtypeStruct((_HEADS, _N, _HID), f32),
                   jax.ShapeDtypeStruct((_HEADS, _N), f32),
                   jax.ShapeDtypeStruct((_N, _HEADS), f32),
                   jax.ShapeDtypeStruct((_N, _HID), f32),
                   jax.ShapeDtypeStruct((_N_CIR, _HID), f32),
                   jax.ShapeDtypeStruct((_HEADS, _N_CIR, _HID), f32),
                   jax.ShapeDtypeStruct((_HEADS, _N_CIR), f32),
                   jax.ShapeDtypeStruct((_N_CIR, _HEADS), f32),
                   jax.ShapeDtypeStruct((_N_CIR, _HID), f32)],
    )(x, x_cir,
      p['proj_W'], p['proj_b'],
      p['conv0']['W'], p['conv0']['a_src'], p['conv0']['a_dst'],
      p['conv0']['W_res'],
      p['proj_cir_W'], p['proj_cir_b'],
      p['convc0']['W'], p['convc0']['a_src'], p['convc0']['a_dst'],
      p['convc0']['W_res'])
    o0, h0, es0, ed0, xr0, o0c, h0c, es0c, ed0c, xrc = pre

    ret, ass = pl.pallas_call(
        _dense_body,
        out_shape=[jax.ShapeDtypeStruct((_N_DRUG, _N_CIR), f32),
                   jax.ShapeDtypeStruct((_N_CIR, _HID), f32)],
    )(c_het, c_cir,
      o0, h0, es0, ed0, xr0, p['conv0']['b'],
      p['conv1']['W'], p['conv1']['a_src'], p['conv1']['a_dst'],
      p['conv1']['b'], p['conv1']['W_res'],
      cnn3, p['cnn_het_b'],
      o0c, h0c, es0c, ed0c, xrc, p['convc0']['b'],
      p['convc1']['W'], p['convc1']['a_src'], p['convc1']['a_dst'],
      p['convc1']['b'], p['convc1']['W_res'],
      cnn3c, p['cnn_cir_b'],
      p['dec_W'])
    return (ret, ass)
